# baseline (speedup 1.0000x reference)
import jax
import jax.numpy as jnp
from jax.experimental import pallas as pl
from jax.experimental.pallas import tpu as pltpu

BEAM = 4
VOCAB = 100000
NKV = 16
NEG_BIG = -1e30


def _fused_body(*refs):
    kv_in = refs[:NKV]
    logits_ref = refs[NKV]
    rp_ref = refs[NKV + 1]
    pen_ref = refs[NKV + 2]
    kv_out = refs[NKV + 3:2 * NKV + 3]
    idx_ref = refs[2 * NKV + 3]
    prob_ref = refs[2 * NKV + 4]
    rp_out_ref = refs[2 * NKV + 5]
    sem = refs[2 * NKV + 6]

    copies = []
    for i in range(NKV):
        for b in range(BEAM):
            c = pltpu.make_async_copy(kv_in[i].at[0], kv_out[i].at[b], sem)
            c.start()
            copies.append(c)

    x = logits_ref[...]
    m = jnp.max(x)
    s = jnp.sum(jnp.exp(x - m))
    lse = m + jnp.log(s)

    col = jax.lax.broadcasted_iota(jnp.int32, (1, VOCAB), 1)
    v = x
    idxs = []
    vals = []
    for _ in range(BEAM):
        mv = jnp.max(v)
        ii = jnp.min(jnp.where(v == mv, col, VOCAB))
        idxs.append(ii)
        vals.append(mv - lse)
        v = jnp.where(col == ii, NEG_BIG, v)

    row = jax.lax.broadcasted_iota(jnp.int32, (BEAM, 1), 0)
    iv = idxs[0]
    pv = vals[0]
    for t in range(1, BEAM):
        iv = jnp.where(row == t, idxs[t], iv)
        pv = jnp.where(row == t, vals[t], pv)
    idx_ref[...] = iv
    prob_ref[...] = pv

    pen = pen_ref[0]
    colb = jax.lax.broadcasted_iota(jnp.int32, (BEAM, VOCAB), 1)
    hit = (colb == idxs[0]) | (colb == idxs[1]) | (colb == idxs[2]) | (colb == idxs[3])
    rp_out_ref[...] = rp_ref[...] * jnp.where(hit, pen, jnp.float32(1.0))

    for c in copies:
        c.wait()


def kernel(kv_0, kv_1, kv_2, kv_3, kv_4, kv_5, kv_6, kv_7, kv_8, kv_9,
           kv_10, kv_11, kv_12, kv_13, kv_14, kv_15,
           logits, save_id, repeat_penality, penality_value, beam_size):
    kvs = [kv_0, kv_1, kv_2, kv_3, kv_4, kv_5, kv_6, kv_7,
           kv_8, kv_9, kv_10, kv_11, kv_12, kv_13, kv_14, kv_15]
    outs = pl.pallas_call(
        _fused_body,
        in_specs=(
            [pl.BlockSpec(memory_space=pl.ANY)] * NKV
            + [pl.BlockSpec(memory_space=pltpu.VMEM),
               pl.BlockSpec(memory_space=pltpu.VMEM),
               pl.BlockSpec(memory_space=pltpu.SMEM)]
        ),
        out_specs=(
            [pl.BlockSpec(memory_space=pl.ANY)] * NKV
            + [pl.BlockSpec(memory_space=pltpu.VMEM)] * 3
        ),
        out_shape=(
            [jax.ShapeDtypeStruct((BEAM, 8, 2048, 64), jnp.float32)] * NKV
            + [jax.ShapeDtypeStruct((BEAM, 1), jnp.int32),
               jax.ShapeDtypeStruct((BEAM, 1), jnp.float32),
               jax.ShapeDtypeStruct((BEAM, VOCAB), jnp.float32)]
        ),
        scratch_shapes=[pltpu.SemaphoreType.DMA],
    )(*kvs, logits, repeat_penality, penality_value)
    saved = outs[:NKV]
    top_idx, top_prob, rp_out = outs[NKV], outs[NKV + 1], outs[NKV + 2]
    beam = save_id.shape[0]
    save_id_out = jnp.concatenate([save_id, top_idx], axis=-1)
    batch_indices = jnp.arange(beam, dtype=jnp.int32) + (
        jnp.asarray(beam_size, dtype=jnp.int32) - beam)
    max_logits_idx = top_idx[0]
    return (*saved, top_idx, save_id_out, rp_out, top_prob,
            batch_indices, max_logits_idx)


# speedup vs baseline: 19.2116x; 19.2116x over previous
import jax
import jax.numpy as jnp
from jax.experimental import pallas as pl
from jax.experimental.pallas import tpu as pltpu

BEAM = 4
VOCAB = 100000
NKV = 16
NEG_BIG = -1e30
KV_ROWS = 8192
NSLOTS = 5


def _topk_compute(logits_ref, rp_ref, pen_ref, idx_ref, prob_ref, rp_out_ref):
    x = logits_ref[...]
    m = jnp.max(x)
    s = jnp.sum(jnp.exp(x - m))
    lse = m + jnp.log(s)

    col = jax.lax.broadcasted_iota(jnp.int32, (1, VOCAB), 1)
    v = x
    idxs = []
    vals = []
    for _ in range(BEAM):
        mv = jnp.max(v)
        ii = jnp.min(jnp.where(v == mv, col, VOCAB))
        idxs.append(ii)
        vals.append(mv - lse)
        v = jnp.where(col == ii, NEG_BIG, v)

    row = jax.lax.broadcasted_iota(jnp.int32, (BEAM, 1), 0)
    iv = idxs[0]
    pv = vals[0]
    for t in range(1, BEAM):
        iv = jnp.where(row == t, idxs[t], iv)
        pv = jnp.where(row == t, vals[t], pv)
    idx_ref[...] = iv
    prob_ref[...] = pv

    pen = pen_ref[0]
    colb = jax.lax.broadcasted_iota(jnp.int32, (BEAM, VOCAB), 1)
    hit = (colb == idxs[0]) | (colb == idxs[1]) | (colb == idxs[2]) | (colb == idxs[3])
    rp_out_ref[...] = rp_ref[...] * jnp.where(hit, pen, jnp.float32(1.0))


def _fused_body(*refs):
    kv_in = refs[:NKV]
    logits_ref = refs[NKV]
    rp_ref = refs[NKV + 1]
    pen_ref = refs[NKV + 2]
    kv_out = refs[NKV + 3:2 * NKV + 3]
    idx_ref = refs[2 * NKV + 3]
    prob_ref = refs[2 * NKV + 4]
    rp_out_ref = refs[2 * NKV + 5]
    vbuf = refs[2 * NKV + 6]
    in_sems = refs[2 * NKV + 7]
    out_sems = refs[2 * NKV + 8]

    def in_copy(i):
        s = i % NSLOTS
        return pltpu.make_async_copy(kv_in[i], vbuf.at[s], in_sems.at[s])

    def out_copy(i, b):
        s = i % NSLOTS
        return pltpu.make_async_copy(vbuf.at[s], kv_out[i].at[b], out_sems.at[s])

    def wait_outs(i):
        for b in range(BEAM):
            out_copy(i, b).wait()

    for i in range(3):
        in_copy(i).start()

    in_copy(0).wait()
    for b in range(BEAM):
        out_copy(0, b).start()

    _topk_compute(logits_ref, rp_ref, pen_ref, idx_ref, prob_ref, rp_out_ref)

    outs_waited = set()
    for i in range(1, NKV):
        in_copy(i).wait()
        for b in range(BEAM):
            out_copy(i, b).start()
        k = i + 2
        if k < NKV:
            if k - NSLOTS >= 0:
                wait_outs(k - NSLOTS)
                outs_waited.add(k - NSLOTS)
            in_copy(k).start()
    for i in range(NKV):
        if i not in outs_waited:
            wait_outs(i)


def kernel(kv_0, kv_1, kv_2, kv_3, kv_4, kv_5, kv_6, kv_7, kv_8, kv_9,
           kv_10, kv_11, kv_12, kv_13, kv_14, kv_15,
           logits, save_id, repeat_penality, penality_value, beam_size):
    kvs = [kv_0, kv_1, kv_2, kv_3, kv_4, kv_5, kv_6, kv_7,
           kv_8, kv_9, kv_10, kv_11, kv_12, kv_13, kv_14, kv_15]
    flat = [kv.reshape(KV_ROWS, 128) for kv in kvs]
    outs = pl.pallas_call(
        _fused_body,
        in_specs=(
            [pl.BlockSpec(memory_space=pl.ANY)] * NKV
            + [pl.BlockSpec(memory_space=pltpu.VMEM),
               pl.BlockSpec(memory_space=pltpu.VMEM),
               pl.BlockSpec(memory_space=pltpu.SMEM)]
        ),
        out_specs=(
            [pl.BlockSpec(memory_space=pl.ANY)] * NKV
            + [pl.BlockSpec(memory_space=pltpu.VMEM)] * 3
        ),
        out_shape=(
            [jax.ShapeDtypeStruct((BEAM, KV_ROWS, 128), jnp.float32)] * NKV
            + [jax.ShapeDtypeStruct((BEAM, 1), jnp.int32),
               jax.ShapeDtypeStruct((BEAM, 1), jnp.float32),
               jax.ShapeDtypeStruct((BEAM, VOCAB), jnp.float32)]
        ),
        scratch_shapes=[
            pltpu.VMEM((NSLOTS, KV_ROWS, 128), jnp.float32),
            pltpu.SemaphoreType.DMA((NSLOTS,)),
            pltpu.SemaphoreType.DMA((NSLOTS,)),
        ],
    )(*flat, logits, repeat_penality, penality_value)
    saved = [o.reshape(BEAM, 8, 2048, 64) for o in outs[:NKV]]
    top_idx, top_prob, rp_out = outs[NKV], outs[NKV + 1], outs[NKV + 2]
    beam = save_id.shape[0]
    save_id_out = jnp.concatenate([save_id, top_idx], axis=-1)
    batch_indices = jnp.arange(beam, dtype=jnp.int32) + (
        jnp.asarray(beam_size, dtype=jnp.int32) - beam)
    max_logits_idx = top_idx[0]
    return (*saved, top_idx, save_id_out, rp_out, top_prob,
            batch_indices, max_logits_idx)


# speedup vs baseline: 24.4424x; 1.2723x over previous
import jax
import jax.numpy as jnp
from jax.experimental import pallas as pl
from jax.experimental.pallas import tpu as pltpu

BEAM = 4
VOCAB = 100000
NKV = 16
NEG_BIG = -1e30
KV_ROWS = 8192
NSLOTS = 5


def _topk_compute(logits_ref, rp_ref, pen_ref, idx_ref, prob_ref, rp_out_ref):
    x = logits_ref[...]
    m = jnp.max(x)
    s = jnp.sum(jnp.exp(x - m))
    lse = m + jnp.log(s)

    col = jax.lax.broadcasted_iota(jnp.int32, (1, VOCAB), 1)
    v = x
    idxs = []
    vals = []
    for _ in range(BEAM):
        mv = jnp.max(v)
        ii = jnp.min(jnp.where(v == mv, col, VOCAB))
        idxs.append(ii)
        vals.append(mv - lse)
        v = jnp.where(col == ii, NEG_BIG, v)

    row = jax.lax.broadcasted_iota(jnp.int32, (BEAM, 1), 0)
    iv = idxs[0]
    pv = vals[0]
    for t in range(1, BEAM):
        iv = jnp.where(row == t, idxs[t], iv)
        pv = jnp.where(row == t, vals[t], pv)
    idx_ref[...] = iv
    prob_ref[...] = pv

    pen = pen_ref[0]
    colb = jax.lax.broadcasted_iota(jnp.int32, (BEAM, VOCAB), 1)
    hit = (colb == idxs[0]) | (colb == idxs[1]) | (colb == idxs[2]) | (colb == idxs[3])
    rp_out_ref[...] = rp_ref[...] * jnp.where(hit, pen, jnp.float32(1.0))


def _fused_body(*refs):
    kv_in = refs[:NKV]
    logits_ref = refs[NKV]
    rp_ref = refs[NKV + 1]
    pen_ref = refs[NKV + 2]
    kv_out = refs[NKV + 3:2 * NKV + 3]
    idx_ref = refs[2 * NKV + 3]
    prob_ref = refs[2 * NKV + 4]
    rp_out_ref = refs[2 * NKV + 5]
    vbuf = refs[2 * NKV + 6]
    in_sems = refs[2 * NKV + 7]
    out_sems = refs[2 * NKV + 8]

    def in_copy(i):
        s = i % NSLOTS
        return pltpu.make_async_copy(kv_in[i].at[0], vbuf.at[s], in_sems.at[s])

    def out_copy(i, b):
        s = i % NSLOTS
        return pltpu.make_async_copy(vbuf.at[s], kv_out[i].at[b], out_sems.at[s])

    def wait_outs(i):
        for b in range(BEAM):
            out_copy(i, b).wait()

    for i in range(3):
        in_copy(i).start()

    in_copy(0).wait()
    for b in range(BEAM):
        out_copy(0, b).start()

    _topk_compute(logits_ref, rp_ref, pen_ref, idx_ref, prob_ref, rp_out_ref)

    outs_waited = set()
    for i in range(1, NKV):
        in_copy(i).wait()
        for b in range(BEAM):
            out_copy(i, b).start()
        k = i + 2
        if k < NKV:
            if k - NSLOTS >= 0:
                wait_outs(k - NSLOTS)
                outs_waited.add(k - NSLOTS)
            in_copy(k).start()
    for i in range(NKV):
        if i not in outs_waited:
            wait_outs(i)


def kernel(kv_0, kv_1, kv_2, kv_3, kv_4, kv_5, kv_6, kv_7, kv_8, kv_9,
           kv_10, kv_11, kv_12, kv_13, kv_14, kv_15,
           logits, save_id, repeat_penality, penality_value, beam_size):
    kvs = [kv_0, kv_1, kv_2, kv_3, kv_4, kv_5, kv_6, kv_7,
           kv_8, kv_9, kv_10, kv_11, kv_12, kv_13, kv_14, kv_15]
    outs = pl.pallas_call(
        _fused_body,
        in_specs=(
            [pl.BlockSpec(memory_space=pl.ANY)] * NKV
            + [pl.BlockSpec(memory_space=pltpu.VMEM),
               pl.BlockSpec(memory_space=pltpu.VMEM),
               pl.BlockSpec(memory_space=pltpu.SMEM)]
        ),
        out_specs=(
            [pl.BlockSpec(memory_space=pl.ANY)] * NKV
            + [pl.BlockSpec(memory_space=pltpu.VMEM)] * 3
        ),
        out_shape=(
            [jax.ShapeDtypeStruct((BEAM, 8, 2048, 64), jnp.float32)] * NKV
            + [jax.ShapeDtypeStruct((BEAM, 1), jnp.int32),
               jax.ShapeDtypeStruct((BEAM, 1), jnp.float32),
               jax.ShapeDtypeStruct((BEAM, VOCAB), jnp.float32)]
        ),
        scratch_shapes=[
            pltpu.VMEM((NSLOTS, 8, 2048, 64), jnp.float32),
            pltpu.SemaphoreType.DMA((NSLOTS,)),
            pltpu.SemaphoreType.DMA((NSLOTS,)),
        ],
    )(*kvs, logits, repeat_penality, penality_value)
    saved = list(outs[:NKV])
    top_idx, top_prob, rp_out = outs[NKV], outs[NKV + 1], outs[NKV + 2]
    beam = save_id.shape[0]
    save_id_out = jnp.concatenate([save_id, top_idx], axis=-1)
    batch_indices = jnp.arange(beam, dtype=jnp.int32) + (
        jnp.asarray(beam_size, dtype=jnp.int32) - beam)
    max_logits_idx = top_idx[0]
    return (*saved, top_idx, save_id_out, rp_out, top_prob,
            batch_indices, max_logits_idx)
